# baseline (speedup 1.0000x reference)
import jax
import jax.numpy as jnp
from jax.experimental import pallas as pl
from jax.experimental.pallas import tpu as pltpu

_OUT_ACTIONS = 18
_BLOCK_B = 5120


def _mlp_kernel(x_ref, w1_ref, b1_ref, w2_ref, b2_ref, o_ref):
    x = x_ref[...].astype(jnp.bfloat16)
    w1 = w1_ref[...].astype(jnp.bfloat16)
    h = jnp.dot(x, w1, preferred_element_type=jnp.float32)
    h = jnp.maximum(h + b1_ref[...], 0.0).astype(jnp.bfloat16)
    w2 = w2_ref[...].astype(jnp.bfloat16)
    y = jnp.dot(h, w2, preferred_element_type=jnp.float32)
    y = y + b2_ref[...]
    o_ref[...] = y[:, :_OUT_ACTIONS]


@jax.jit
def kernel(x, w1, b1, w2, b2):
    B, K = x.shape
    Hp = w1.shape[1]
    Np = w2.shape[1]
    block_b = min(_BLOCK_B, B)
    nb = pl.cdiv(B, block_b)
    flops = 2 * B * (K * Hp + Hp * Np)
    w_bytes = (w1.size + b1.size + w2.size + b2.size) * 4
    cost = pl.CostEstimate(
        flops=flops, transcendentals=0,
        bytes_accessed=B * K * 4 + w_bytes + B * _OUT_ACTIONS * 4)
    return pl.pallas_call(
        _mlp_kernel,
        out_shape=jax.ShapeDtypeStruct((B, _OUT_ACTIONS), jnp.float32),
        grid=(nb,),
        in_specs=[
            pl.BlockSpec((block_b, K), lambda i: (i, 0)),
            pl.BlockSpec((K, Hp), lambda i: (0, 0)),
            pl.BlockSpec((1, Hp), lambda i: (0, 0)),
            pl.BlockSpec((Hp, Np), lambda i: (0, 0)),
            pl.BlockSpec((1, Np), lambda i: (0, 0)),
        ],
        out_specs=pl.BlockSpec((block_b, _OUT_ACTIONS), lambda i: (i, 0)),
        compiler_params=pltpu.CompilerParams(
            dimension_semantics=("parallel",)),
        cost_estimate=cost,
    )(x, w1, b1, w2, b2)


# speedup vs baseline: 1.1135x; 1.1135x over previous
import jax
import jax.numpy as jnp
from jax.experimental import pallas as pl
from jax.experimental.pallas import tpu as pltpu

_OUT_ACTIONS = 18
_BLOCK_B = 4096


def _pipelined_kernel(x_ref, w1_ref, b1_ref, w2_ref, b2_ref, o_ref, h_ref):
    i = pl.program_id(0)
    n = pl.num_programs(0)

    @pl.when(i > 0)
    def _fc2():
        h = h_ref[(i + 1) % 2]
        w2 = w2_ref[...].astype(jnp.bfloat16)
        y = jnp.dot(h, w2, preferred_element_type=jnp.float32)
        o_ref[...] = (y + b2_ref[...])[:, :_OUT_ACTIONS]

    @pl.when(i < n - 1)
    def _fc1():
        x = x_ref[...].astype(jnp.bfloat16)
        w1 = w1_ref[...].astype(jnp.bfloat16)
        h = jnp.dot(x, w1, preferred_element_type=jnp.float32)
        h_ref[i % 2] = jnp.maximum(h + b1_ref[...], 0.0).astype(jnp.bfloat16)


@jax.jit
def kernel(x, w1, b1, w2, b2):
    B, K = x.shape
    Hp = w1.shape[1]
    Np = w2.shape[1]
    block_b = min(_BLOCK_B, B)
    nb = pl.cdiv(B, block_b)
    flops = 2 * B * (K * Hp + Hp * Np)
    w_bytes = (w1.size + b1.size + w2.size + b2.size) * 4
    cost = pl.CostEstimate(
        flops=flops, transcendentals=0,
        bytes_accessed=B * K * 4 + w_bytes + B * _OUT_ACTIONS * 4)
    return pl.pallas_call(
        _pipelined_kernel,
        out_shape=jax.ShapeDtypeStruct((B, _OUT_ACTIONS), jnp.float32),
        grid=(nb + 1,),
        in_specs=[
            pl.BlockSpec((block_b, K), lambda i: (jnp.minimum(i, nb - 1), 0)),
            pl.BlockSpec((K, Hp), lambda i: (0, 0)),
            pl.BlockSpec((1, Hp), lambda i: (0, 0)),
            pl.BlockSpec((Hp, Np), lambda i: (0, 0)),
            pl.BlockSpec((1, Np), lambda i: (0, 0)),
        ],
        out_specs=pl.BlockSpec(
            (block_b, _OUT_ACTIONS), lambda i: (jnp.maximum(i - 1, 0), 0)),
        scratch_shapes=[
            pltpu.VMEM((2, block_b, Hp), jnp.bfloat16),
        ],
        compiler_params=pltpu.CompilerParams(
            dimension_semantics=("arbitrary",)),
        cost_estimate=cost,
    )(x, w1, b1, w2, b2)
